# baseline (speedup 1.0000x reference)
import math

import jax
import jax.numpy as jnp
import numpy as np
from jax import lax
from jax.experimental import pallas as pl
from jax.experimental.pallas import tpu as pltpu
from jax.experimental.pallas import tpu_sc as plsc

_SPIKE_THRESHOLD = math.log(0.9)
_BLANK_CUT = float(np.float32(1.1053606))
_B, _T, _V = 16, 2048, 512
_BPG = 16


def _tc_body(tl_ref, alpha_ref, ctc_ref, mask_ref, out_ref, cacc):
    g = pl.program_id(0)

    for j in range(_BPG):
        b = g * _BPG + j
        x = ctc_ref[j]
        trig = (x < _BLANK_CUT).astype(jnp.bfloat16)
        m01 = (mask_ref[pl.ds(b, 1), :] != 0.0).astype(jnp.bfloat16)
        y = jax.lax.dot_general(m01, trig, (((1,), (0,)), ((), ())),
                                preferred_element_type=jnp.float32)
        cacc[pl.ds(b, 1), :] = y

    @pl.when(g == _B // _BPG - 1)
    def _final():
        lane = lax.broadcasted_iota(jnp.int32, (1, 128), 1)
        e0 = (lane == 0).astype(jnp.float32)
        counts = jax.lax.dot_general(
            e0, cacc[...], (((1,), (1,)), ((), ())),
            preferred_element_type=jnp.float32)
        ones_t = jnp.ones((1, _T), jnp.float32)
        rvs = jax.lax.dot_general(
            ones_t, alpha_ref[...], (((1,), (1,)), ((), ())),
            preferred_element_type=jnp.float32)
        lanes_b = lax.broadcasted_iota(jnp.int32, (1, _B), 1)
        lt = jnp.zeros((1, _B), jnp.float32)
        for i in range(_B):
            lt += jnp.where(lanes_b == i, tl_ref[i].astype(jnp.float32), 0.0)
        has = counts >= 1.0
        n = jnp.where(has, counts, 1.0)
        rv = jnp.where(has, rvs, 1.0)
        max_s = jnp.max(n)
        max_len = jnp.maximum(1.0, jnp.max(lt))
        length = jnp.minimum(max_s, max_len)
        l_b = jnp.minimum(lt, length)
        m_b = jnp.minimum(l_b, n)
        contrib = jnp.abs(rv - 1.0) * m_b + (l_b - m_b)
        out_ref[0, 0] = jnp.sum(contrib) * (1.0 / _B)


_BSC = 4
_BTC = _B - _BSC
_TPB = 32 // _BSC
_TPW = _T // _TPB
_CH = 128
_NCHS = _TPW // _CH


def _sc_stats_body(ctc_hbm, mask_hbm, out_hbm,
                   buf0, buf1, mask_v, cnt_st, sem0, sem1):
    c = lax.axis_index("c")
    s = lax.axis_index("s")
    w = c * 16 + s
    b = _BTC + w // _TPB
    t0 = (w % _TPB) * _TPW

    pltpu.sync_copy(mask_hbm.at[b, pl.ds(t0, _TPW)],
                    mask_v.at[pl.ds(0, _TPW)])

    iota = lax.iota(jnp.int32, 16)
    e0f = jnp.where(iota == 0, 1.0, 0.0)
    zeros = jnp.zeros((16,), jnp.float32)
    bufs = (buf0, buf1)
    sems = (sem0, sem1)
    handles = [None, None]
    handles[0] = pltpu.async_copy(
        ctc_hbm.at[b, pl.ds(t0, _CH), pl.ds(0, 128)], buf0, sem0)

    cnt_vec = jnp.zeros((16,), jnp.float32)
    for k in range(_NCHS):
        if k + 1 < _NCHS:
            handles[(k + 1) % 2] = pltpu.async_copy(
                ctc_hbm.at[b, pl.ds(t0 + (k + 1) * _CH, _CH), pl.ds(0, 128)],
                bufs[(k + 1) % 2], sems[(k + 1) % 2])
        handles[k % 2].wait()
        buf = bufs[k % 2]

        def grp(g, cnt, buf=buf, k=k):
            base_r = g * 16
            gidx0 = k * _CH + base_r
            for j in range(16):
                rowvec = buf[base_r + j, pl.ds(0, 16)]
                mrow = mask_v[pl.ds(gidx0 + j, 16)]
                hit = jnp.logical_and(rowvec < _BLANK_CUT, mrow != 0.0)
                cnt = cnt + jnp.where(hit, e0f, zeros)
            return cnt

        cnt_vec = lax.fori_loop(0, _CH // 16, grp, cnt_vec)

    cnt_st[...] = cnt_vec
    pltpu.sync_copy(cnt_st, out_hbm.at[w])


def _sc_stats(ctc_log_probs, mask):
    mesh = plsc.VectorSubcoreMesh(core_axis_name="c", subcore_axis_name="s")
    call = pl.kernel(
        _sc_stats_body,
        out_type=jax.ShapeDtypeStruct((32, 16), jnp.float32),
        mesh=mesh,
        scratch_types=[
            pltpu.VMEM((_CH, 128), jnp.float32),
            pltpu.VMEM((_CH, 128), jnp.float32),
            pltpu.VMEM((_TPW + 16,), jnp.float32),
            pltpu.VMEM((16,), jnp.float32),
            pltpu.SemaphoreType.DMA,
            pltpu.SemaphoreType.DMA,
        ],
    )
    return call(ctc_log_probs, mask)


def _tc_stats_body(ctc_ref, mask_ref, out_ref):
    g = pl.program_id(0)
    for j in range(_BPG):
        b = g * _BPG + j
        x = ctc_ref[j]
        trig = (x < _BLANK_CUT).astype(jnp.bfloat16)
        m01 = (mask_ref[pl.ds(b, 1), :] != 0.0).astype(jnp.bfloat16)
        y = jax.lax.dot_general(m01, trig, (((1,), (0,)), ((), ())),
                                preferred_element_type=jnp.float32)
        out_ref[pl.ds(b, 1), :] = y


def _tc_stats(ctc_log_probs, mask):
    return pl.pallas_call(
        _tc_stats_body,
        grid=(_BTC // _BPG,),
        in_specs=[
            pl.BlockSpec((_BPG, _T, 128), lambda g: (g, 0, 0)),
            pl.BlockSpec((_B, _T), lambda g: (0, 0)),
        ],
        out_specs=pl.BlockSpec((_B, 128), lambda g: (0, 0)),
        out_shape=jax.ShapeDtypeStruct((_B, 128), jnp.float32),
    )(ctc_log_probs, mask)


def _comb_body(tl_ref, alpha_ref, tcc_ref, st_ref, out_ref):
    lane128 = lax.broadcasted_iota(jnp.int32, (1, 128), 1)
    e0_128 = (lane128 == 0).astype(jnp.float32)
    counts_tc = jax.lax.dot_general(
        e0_128, tcc_ref[...], (((1,), (1,)), ((), ())),
        preferred_element_type=jnp.float32)

    x0 = st_ref[...]
    lane16 = lax.broadcasted_iota(jnp.int32, (1, 16), 1)
    e0_16 = (lane16 == 0).astype(jnp.float32)
    c1 = jax.lax.dot_general(e0_16, x0, (((1,), (1,)), ((), ())),
                             preferred_element_type=jnp.float32)
    wv = lax.broadcasted_iota(jnp.int32, (32, 16), 0)
    bv = lax.broadcasted_iota(jnp.int32, (32, 16), 1)
    amat = ((_BTC + wv // _TPB) == bv).astype(jnp.float32)
    counts_sc = jax.lax.dot_general(c1, amat, (((1,), (0,)), ((), ())),
                                    preferred_element_type=jnp.float32)

    ones_t = jnp.ones((1, _T), jnp.float32)
    rvs = jax.lax.dot_general(
        ones_t, alpha_ref[...], (((1,), (1,)), ((), ())),
        preferred_element_type=jnp.float32)

    lanes_b = lax.broadcasted_iota(jnp.int32, (1, _B), 1)
    counts = jnp.where(lanes_b < _BTC, counts_tc, counts_sc)
    lt = jnp.zeros((1, _B), jnp.float32)
    for i in range(_B):
        lt += jnp.where(lanes_b == i, tl_ref[i].astype(jnp.float32), 0.0)
    has = counts >= 1.0
    n = jnp.where(has, counts, 1.0)
    rv = jnp.where(has, rvs, 1.0)
    max_s = jnp.max(n)
    max_len = jnp.maximum(1.0, jnp.max(lt))
    length = jnp.minimum(max_s, max_len)
    l_b = jnp.minimum(lt, length)
    m_b = jnp.minimum(l_b, n)
    contrib = jnp.abs(rv - 1.0) * m_b + (l_b - m_b)
    out_ref[0, 0] = jnp.sum(contrib) * (1.0 / _B)


def _combine(text_length, alpha, tc_stats, sc_stats):
    out = pl.pallas_call(
        _comb_body,
        in_specs=[
            pl.BlockSpec(memory_space=pltpu.SMEM),
            pl.BlockSpec((_B, _T), lambda: (0, 0)),
            pl.BlockSpec((_B, 128), lambda: (0, 0)),
            pl.BlockSpec((32, 16), lambda: (0, 0)),
        ],
        out_specs=pl.BlockSpec(memory_space=pltpu.SMEM),
        out_shape=jax.ShapeDtypeStruct((1, 1), jnp.float32),
    )(text_length, alpha, tc_stats, sc_stats)
    return out[0, 0]


@jax.jit
def _sc_loss(alpha, ctc_log_probs, mask, text_length):
    sc = _sc_stats(ctc_log_probs, mask)
    tc = _tc_stats(ctc_log_probs, mask)
    return _combine(text_length, alpha, tc, sc)


@jax.jit
def _tc_loss(alpha, ctc_log_probs, mask, text_length):
    out = pl.pallas_call(
        _tc_body,
        grid=(_B // _BPG,),
        in_specs=[
            pl.BlockSpec(memory_space=pltpu.SMEM),
            pl.BlockSpec((_B, _T), lambda g: (0, 0)),
            pl.BlockSpec((_BPG, _T, 128), lambda g: (g, 0, 0)),
            pl.BlockSpec((_B, _T), lambda g: (0, 0)),
        ],
        out_specs=pl.BlockSpec(memory_space=pltpu.SMEM),
        out_shape=jax.ShapeDtypeStruct((1, 1), jnp.float32),
        scratch_shapes=[
            pltpu.VMEM((_B, 128), jnp.float32),
        ],
    )(text_length, alpha, ctc_log_probs, mask)
    return out[0, 0]


def kernel(alpha, ctc_log_probs, mask, text_length):
    return _tc_loss(alpha, ctc_log_probs, mask, text_length)


# speedup vs baseline: 1.0784x; 1.0784x over previous
import math

import jax
import jax.numpy as jnp
import numpy as np
from jax import lax
from jax.experimental import pallas as pl
from jax.experimental.pallas import tpu as pltpu

_SPIKE_THRESHOLD = math.log(0.9)
_BLANK_CUT = float(np.float32(1.1053606))
_B, _T, _V = 16, 2048, 512
_BPG = 8


def _tc_body(tl_ref, alpha_ref, ctc_ref, mask_ref, out_ref, cacc):
    g = pl.program_id(0)

    for j in range(_BPG):
        b = g * _BPG + j
        x = ctc_ref[j]
        trig = (x < _BLANK_CUT).astype(jnp.bfloat16)
        m01 = (mask_ref[pl.ds(b, 1), :] != 0.0).astype(jnp.bfloat16)
        y = jax.lax.dot_general(m01, trig, (((1,), (0,)), ((), ())),
                                preferred_element_type=jnp.float32)
        cacc[pl.ds(b, 1), :] = y

    @pl.when(g == _B // _BPG - 1)
    def _final():
        lane = lax.broadcasted_iota(jnp.int32, (1, 128), 1)
        e0 = (lane == 0).astype(jnp.float32)
        counts = jax.lax.dot_general(
            e0, cacc[...], (((1,), (1,)), ((), ())),
            preferred_element_type=jnp.float32)
        ones_t = jnp.ones((1, _T), jnp.float32)
        rvs = jax.lax.dot_general(
            ones_t, alpha_ref[...], (((1,), (1,)), ((), ())),
            preferred_element_type=jnp.float32)
        lanes_b = lax.broadcasted_iota(jnp.int32, (1, _B), 1)
        lt = jnp.zeros((1, _B), jnp.float32)
        for i in range(_B):
            lt += jnp.where(lanes_b == i, tl_ref[i].astype(jnp.float32), 0.0)
        has = counts >= 1.0
        n = jnp.where(has, counts, 1.0)
        rv = jnp.where(has, rvs, 1.0)
        max_s = jnp.max(n)
        max_len = jnp.maximum(1.0, jnp.max(lt))
        length = jnp.minimum(max_s, max_len)
        l_b = jnp.minimum(lt, length)
        m_b = jnp.minimum(l_b, n)
        contrib = jnp.abs(rv - 1.0) * m_b + (l_b - m_b)
        out_ref[0, 0] = jnp.sum(contrib) * (1.0 / _B)


@jax.jit
def _tc_loss(alpha, ctc_log_probs, mask, text_length):
    out = pl.pallas_call(
        _tc_body,
        grid=(_B // _BPG,),
        in_specs=[
            pl.BlockSpec(memory_space=pltpu.SMEM),
            pl.BlockSpec((_B, _T), lambda g: (0, 0)),
            pl.BlockSpec((_BPG, _T, 128), lambda g: (g, 0, 0)),
            pl.BlockSpec((_B, _T), lambda g: (0, 0)),
        ],
        out_specs=pl.BlockSpec(memory_space=pltpu.SMEM),
        out_shape=jax.ShapeDtypeStruct((1, 1), jnp.float32),
        scratch_shapes=[
            pltpu.VMEM((_B, 128), jnp.float32),
        ],
    )(text_length, alpha, ctc_log_probs, mask)
    return out[0, 0]


def kernel(alpha, ctc_log_probs, mask, text_length):
    return _tc_loss(alpha, ctc_log_probs, mask, text_length)
